# baseline (speedup 1.0000x reference)
import functools

import jax
import jax.numpy as jnp
from jax import lax
from jax.experimental import pallas as pl
from jax.experimental.pallas import tpu as pltpu
from jax.experimental.pallas import tpu_sc as plsc

_NC, _NS = 2, 16
_NW = _NC * _NS
_C = 128
_RING = 10


@functools.cache
def _make_kernel(n: int, d: int):
    per_w = n // _NW
    chunks = per_w // _C
    assert per_w % _C == 0 and chunks % _RING == 0
    mesh = plsc.VectorSubcoreMesh(
        core_axis_name="c", subcore_axis_name="s",
        num_cores=_NC, num_subcores=_NS,
    )

    @functools.partial(
        pl.kernel,
        out_type=jax.ShapeDtypeStruct((n, d), jnp.float32),
        mesh=mesh,
        scratch_types=[
            pltpu.VMEM((chunks, _C), jnp.int32),
            pltpu.VMEM((_RING, _C, d), jnp.float32),
        ] + [pltpu.SemaphoreType.DMA] * (2 * _RING),
        compiler_params=pltpu.CompilerParams(use_tc_tiling_on_sc=False),
    )
    def k(idx_hbm, table_hbm, out_hbm, idx_v, bufs, *sems):
        gsem, wsem = sems[:_RING], sems[_RING:]
        wid = lax.axis_index("s") * _NC + lax.axis_index("c")
        row0 = wid * per_w
        pltpu.sync_copy(idx_hbm.at[wid], idx_v)

        for b in range(_RING):
            pltpu.async_copy(table_hbm.at[idx_v.at[b]], bufs.at[b], gsem[b])

        @pl.loop(0, chunks, step=_RING)
        def _(j0):
            for b in range(_RING):
                j = j0 + b
                pltpu.make_async_copy(
                    table_hbm.at[idx_v.at[j]], bufs.at[b], gsem[b]).wait()
                pltpu.async_copy(
                    bufs.at[b], out_hbm.at[pl.ds(row0 + j * _C, _C)], wsem[b])
                j2 = j + _RING

                @pl.when(j2 < chunks)
                def _():
                    pltpu.make_async_copy(
                        bufs.at[b], out_hbm.at[pl.ds(row0 + j * _C, _C)],
                        wsem[b]).wait()
                    pltpu.async_copy(
                        table_hbm.at[idx_v.at[j2]], bufs.at[b], gsem[b])

        for b in range(_RING):
            j = chunks - _RING + b
            pltpu.make_async_copy(
                bufs.at[b], out_hbm.at[pl.ds(row0 + j * _C, _C)],
                wsem[b]).wait()

    return k


def kernel(indices, table):
    b, h = indices.shape
    _, d = table.shape
    n = b * h
    per_w = n // _NW
    idx = indices.astype(jnp.int32).reshape(_NW, per_w // _C, _C)
    out = _make_kernel(n, d)(idx, table)
    return out.reshape(b, h, d)
